# baseline (speedup 1.0000x reference)
import functools

import jax
import jax.numpy as jnp
from jax import lax
from jax.experimental import pallas as pl
from jax.experimental.pallas import tpu as pltpu
from jax.experimental.pallas import tpu_sc as plsc

V = 1000
D = 1000
DP = 1024
T = 50
TP = 56
C0 = 896
TW = D - C0
NB = 32
NC, NS = 2, 16
NW = NC * NS


W = 64


@functools.partial(jax.jit, static_argnames=("B",))
def _gather_main(table_p, idx_p, B):
    rows = B * TP
    r_per_w = rows // NW
    n_chunks = r_per_w // W
    mesh = plsc.VectorSubcoreMesh(core_axis_name="c", subcore_axis_name="s")

    @functools.partial(
        pl.kernel,
        mesh=mesh,
        out_type=jax.ShapeDtypeStruct((rows, DP), jnp.float32),
        scratch_types=[
            pltpu.VMEM((r_per_w,), jnp.int32),
            pltpu.VMEM((W, DP), jnp.float32),
            pltpu.SemaphoreType.DMA,
        ],
    )
    def k(table_hbm, idx_hbm, out_hbm, idx_v, buf, g):
        wid = lax.axis_index("s") * NC + lax.axis_index("c")
        base = wid * r_per_w
        pltpu.sync_copy(idx_hbm.at[pl.ds(base, r_per_w)], idx_v)

        @pl.loop(0, n_chunks)
        def _(c):
            off = c * W
            pltpu.async_copy(
                table_hbm.at[idx_v.at[pl.ds(off, W)]], buf, g
            ).wait()
            pltpu.sync_copy(buf, out_hbm.at[pl.ds(base + off, W)])

    return k(table_p, idx_p)


def _merge_body(y_ref, idx_ref, hi_ref, lo_ref, out_ref):
    idxb = idx_ref[...]
    oh = (
        idxb[:, :, None]
        == lax.broadcasted_iota(jnp.int32, (NB, T, V), 2)
    ).astype(jnp.bfloat16)
    dims = (((2,), (0,)), ((), ()))
    acc = lax.dot_general(
        oh, hi_ref[...], dims, preferred_element_type=jnp.float32
    )
    acc += lax.dot_general(
        oh, lo_ref[...], dims, preferred_element_type=jnp.float32
    )
    out_ref[:, :, :C0] = y_ref[:, :T, :C0]
    out_ref[:, :, C0:] = acc[:, :, :TW]


@jax.jit
def _merge(y, idx, tail_hi, tail_lo):
    B = y.shape[0]
    return pl.pallas_call(
        _merge_body,
        grid=(B // NB,),
        in_specs=[
            pl.BlockSpec((NB, TP, DP), lambda i: (i, 0, 0)),
            pl.BlockSpec((NB, T), lambda i: (i, 0)),
            pl.BlockSpec((V, 128), lambda i: (0, 0)),
            pl.BlockSpec((V, 128), lambda i: (0, 0)),
        ],
        out_specs=pl.BlockSpec((NB, T, D), lambda i: (i, 0, 0)),
        out_shape=jax.ShapeDtypeStruct((B, T, D), jnp.float32),
    )(y, idx, tail_hi, tail_lo)


def kernel(idx, token_emb):
    B, T_ = idx.shape
    idx32 = idx.astype(jnp.int32)
    idx_p = jnp.pad(idx32, ((0, 0), (0, TP - T))).reshape(-1)
    table_p = jnp.pad(token_emb, ((0, 0), (0, DP - D)))
    tail = jnp.pad(token_emb[:, C0:], ((0, 0), (0, 128 - TW)))
    tail_hi = tail.astype(jnp.bfloat16)
    tail_lo = (tail - tail_hi.astype(jnp.float32)).astype(jnp.bfloat16)
    y = _gather_main(table_p, idx_p, B).reshape(B, TP, DP)
    return _merge(y, idx32, tail_hi, tail_lo)


# speedup vs baseline: 1.7918x; 1.7918x over previous
import functools

import jax
import jax.numpy as jnp
from jax import lax
from jax.experimental import pallas as pl
from jax.experimental.pallas import tpu as pltpu
from jax.experimental.pallas import tpu_sc as plsc

V = 1000
D = 1000
DP = 1024
T = 50
TP = 56
C0 = 896
TW = D - C0
NB = 32
NC, NS = 2, 16
NW = NC * NS


W = 64


@functools.partial(jax.jit, static_argnames=("B",))
def _gather_main(table_p, idx_p, B):
    rows = B * TP
    r_per_w = rows // NW
    n_chunks = r_per_w // W
    mesh = plsc.VectorSubcoreMesh(core_axis_name="c", subcore_axis_name="s")

    @functools.partial(
        pl.kernel,
        mesh=mesh,
        out_type=jax.ShapeDtypeStruct((rows, DP), jnp.float32),
        scratch_types=[
            pltpu.VMEM((r_per_w,), jnp.int32),
            pltpu.VMEM((W, DP), jnp.float32),
            pltpu.SemaphoreType.DMA,
        ],
    )
    def k(table_hbm, idx_hbm, out_hbm, idx_v, buf, g):
        wid = lax.axis_index("s") * NC + lax.axis_index("c")
        base = wid * r_per_w
        pltpu.sync_copy(idx_hbm.at[pl.ds(base, r_per_w)], idx_v)

        @pl.loop(0, n_chunks)
        def _(c):
            off = c * W
            pltpu.async_copy(
                table_hbm.at[idx_v.at[pl.ds(off, W)]], buf, g
            ).wait()
            pltpu.sync_copy(buf, out_hbm.at[pl.ds(base + off, W)])

    return k(table_p, idx_p)


def _merge_body(y_ref, idx_ref, hi_ref, lo_ref, out_ref):
    idxb = idx_ref[...]
    oh = (
        idxb[:, :, None]
        == lax.broadcasted_iota(jnp.int32, (NB, T, V), 2)
    ).astype(jnp.bfloat16)
    dims = (((2,), (0,)), ((), ()))
    acc = lax.dot_general(
        oh, hi_ref[...], dims, preferred_element_type=jnp.float32
    )
    acc += lax.dot_general(
        oh, lo_ref[...], dims, preferred_element_type=jnp.float32
    )
    out_ref[:, :, :C0] = y_ref[:, :T, :C0]
    out_ref[:, :, C0:] = acc[:, :, :TW]


@jax.jit
def _merge(y, idx, tail_hi, tail_lo):
    B = y.shape[0]
    return pl.pallas_call(
        _merge_body,
        grid=(B // NB,),
        in_specs=[
            pl.BlockSpec((NB, TP, DP), lambda i: (i, 0, 0)),
            pl.BlockSpec((NB, T), lambda i: (i, 0)),
            pl.BlockSpec((V, 128), lambda i: (0, 0)),
            pl.BlockSpec((V, 128), lambda i: (0, 0)),
        ],
        out_specs=pl.BlockSpec((NB, T, D), lambda i: (i, 0, 0)),
        out_shape=jax.ShapeDtypeStruct((B, T, D), jnp.float32),
    )(y, idx, tail_hi, tail_lo)


def kernel(idx, token_emb):
    B, T_ = idx.shape
    idx32 = idx.astype(jnp.int32)
    pad_ix = (
        jnp.arange(B, dtype=jnp.int32)[:, None] * (TP - T)
        + jnp.arange(TP - T, dtype=jnp.int32)[None, :]
    ) % V
    idx_p = jnp.concatenate([idx32, pad_ix], axis=1).reshape(-1)
    table_p = jnp.pad(token_emb, ((0, 0), (0, DP - D)))
    tail = jnp.pad(token_emb[:, C0:], ((0, 0), (0, 128 - TW)))
    tail_hi = tail.astype(jnp.bfloat16)
    tail_lo = (tail - tail_hi.astype(jnp.float32)).astype(jnp.bfloat16)
    y = _gather_main(table_p, idx_p, B).reshape(B, TP, DP)
    return _merge(y, idx32, tail_hi, tail_lo)


# speedup vs baseline: 2.0215x; 1.1282x over previous
import functools

import jax
import jax.numpy as jnp
from jax import lax
from jax.experimental import pallas as pl
from jax.experimental.pallas import tpu as pltpu
from jax.experimental.pallas import tpu_sc as plsc

V = 1000
D = 1000
DP = 1024
T = 50
TP = 56
C0 = 896
TW = D - C0
NB = 32
NC, NS = 2, 16
NW = NC * NS


W = 64


@functools.partial(jax.jit, static_argnames=("B",))
def _gather_main(table_p, idx_p, B):
    rows = B * T
    r_per_w = rows // NW
    n_chunks = r_per_w // W
    mesh = plsc.VectorSubcoreMesh(core_axis_name="c", subcore_axis_name="s")

    @functools.partial(
        pl.kernel,
        mesh=mesh,
        out_type=jax.ShapeDtypeStruct((rows, DP), jnp.float32),
        scratch_types=[
            pltpu.VMEM((r_per_w,), jnp.int32),
            pltpu.VMEM((W, DP), jnp.float32),
            pltpu.SemaphoreType.DMA,
        ],
    )
    def k(table_hbm, idx_hbm, out_hbm, idx_v, buf, g):
        wid = lax.axis_index("s") * NC + lax.axis_index("c")
        base = wid * r_per_w
        pltpu.sync_copy(idx_hbm.at[pl.ds(base, r_per_w)], idx_v)

        @pl.loop(0, n_chunks)
        def _(c):
            off = c * W
            pltpu.async_copy(
                table_hbm.at[idx_v.at[pl.ds(off, W)]], buf, g
            ).wait()
            pltpu.sync_copy(buf, out_hbm.at[pl.ds(base + off, W)])

    return k(table_p, idx_p)


def _merge_body(y_ref, idx_ref, hi_ref, lo_ref, out_ref):
    idxb = idx_ref[...]
    oh = (
        idxb[:, :, None]
        == lax.broadcasted_iota(jnp.int32, (NB, T, V), 2)
    ).astype(jnp.bfloat16)
    dims = (((2,), (0,)), ((), ()))
    acc = lax.dot_general(
        oh, hi_ref[...], dims, preferred_element_type=jnp.float32
    )
    acc += lax.dot_general(
        oh, lo_ref[...], dims, preferred_element_type=jnp.float32
    )
    out_ref[:, :, :C0] = y_ref[:, :T, :C0]
    out_ref[:, :, C0:] = acc[:, :, :TW]


@jax.jit
def _merge(y, idx, tail_hi, tail_lo):
    B = y.shape[0]
    return pl.pallas_call(
        _merge_body,
        grid=(B // NB,),
        in_specs=[
            pl.BlockSpec((NB, TP, DP), lambda i: (i, 0, 0)),
            pl.BlockSpec((NB, T), lambda i: (i, 0)),
            pl.BlockSpec((V, 128), lambda i: (0, 0)),
            pl.BlockSpec((V, 128), lambda i: (0, 0)),
        ],
        out_specs=pl.BlockSpec((NB, T, D), lambda i: (i, 0, 0)),
        out_shape=jax.ShapeDtypeStruct((B, T, D), jnp.float32),
    )(y, idx, tail_hi, tail_lo)


def kernel(idx, token_emb):
    B, T_ = idx.shape
    idx32 = idx.astype(jnp.int32)
    table_p = jnp.pad(token_emb, ((0, 0), (0, DP - D)))
    y = _gather_main(table_p, idx32.reshape(-1), B).reshape(B, T, DP)
    return y[:, :, :D]


# speedup vs baseline: 2.9195x; 1.4442x over previous
import functools

import jax
import jax.numpy as jnp
from jax import lax
from jax.experimental import pallas as pl
from jax.experimental.pallas import tpu as pltpu
from jax.experimental.pallas import tpu_sc as plsc

V = 1000
D = 1000
DP = 1024
T = 50
TP = 56
NC, NS = 2, 16
NW = NC * NS
W = 64


@functools.partial(jax.jit, static_argnames=("B",))
def _gather_main(table_p, idx_p, B):
    rows = B * TP
    r_per_w = rows // NW
    n_chunks = r_per_w // W
    mesh = plsc.VectorSubcoreMesh(core_axis_name="c", subcore_axis_name="s")

    @functools.partial(
        pl.kernel,
        mesh=mesh,
        out_type=jax.ShapeDtypeStruct((rows, DP), jnp.float32),
        scratch_types=[
            pltpu.VMEM((r_per_w,), jnp.int32),
            pltpu.VMEM((W, DP), jnp.float32),
            pltpu.SemaphoreType.DMA,
        ],
    )
    def k(table_hbm, idx_hbm, out_hbm, idx_v, buf, g):
        wid = lax.axis_index("s") * NC + lax.axis_index("c")
        base = wid * r_per_w
        pltpu.sync_copy(idx_hbm.at[pl.ds(base, r_per_w)], idx_v)

        @pl.loop(0, n_chunks)
        def _(c):
            off = c * W
            pltpu.async_copy(
                table_hbm.at[idx_v.at[pl.ds(off, W)]], buf, g
            ).wait()
            pltpu.sync_copy(buf, out_hbm.at[pl.ds(base + off, W)])

    return k(table_p, idx_p)


def kernel(idx, token_emb):
    B, T_ = idx.shape
    idx32 = idx.astype(jnp.int32)
    pad_ix = (
        jnp.arange(B, dtype=jnp.int32)[:, None] * (TP - T)
        + jnp.arange(TP - T, dtype=jnp.int32)[None, :]
    ) % V
    idx_p = jnp.concatenate([idx32, pad_ix], axis=1).reshape(-1)
    table_p = jnp.pad(token_emb, ((0, 0), (0, DP - D)))
    y = _gather_main(table_p, idx_p, B).reshape(B, TP, DP)
    return y[:, :T, :D]


# speedup vs baseline: 2.9915x; 1.0247x over previous
import functools

import jax
import jax.numpy as jnp
from jax import lax
from jax.experimental import pallas as pl
from jax.experimental.pallas import tpu as pltpu
from jax.experimental.pallas import tpu_sc as plsc

V = 1000
D = 1000
DP = 1024
T = 50
TP = 56
NC, NS = 2, 16
NW = NC * NS
W = 56


@functools.partial(jax.jit, static_argnames=("B",))
def _gather_main(table_p, idx_p, B):
    rows = B * TP
    r_per_w = rows // NW
    n_chunks = r_per_w // W
    mesh = plsc.VectorSubcoreMesh(core_axis_name="c", subcore_axis_name="s")

    @functools.partial(
        pl.kernel,
        mesh=mesh,
        out_type=jax.ShapeDtypeStruct((rows, DP), jnp.float32),
        scratch_types=[
            pltpu.VMEM((r_per_w,), jnp.int32),
            pltpu.VMEM((W, DP), jnp.float32),
            pltpu.VMEM((W, DP), jnp.float32),
            pltpu.SemaphoreType.DMA,
            pltpu.SemaphoreType.DMA,
            pltpu.SemaphoreType.DMA,
            pltpu.SemaphoreType.DMA,
        ],
    )
    def k(table_hbm, idx_hbm, out_hbm, idx_v, buf0, buf1, g0, g1, w0, w1):
        wid = lax.axis_index("s") * NC + lax.axis_index("c")
        base = wid * r_per_w
        pltpu.sync_copy(idx_hbm.at[pl.ds(base, r_per_w)], idx_v)

        def g_start(c, buf, sem):
            pltpu.async_copy(table_hbm.at[idx_v.at[pl.ds(c * W, W)]], buf, sem)

        def g_wait(c, buf, sem):
            pltpu.make_async_copy(
                table_hbm.at[idx_v.at[pl.ds(c * W, W)]], buf, sem
            ).wait()

        def w_start(c, buf, sem):
            pltpu.async_copy(buf, out_hbm.at[pl.ds(base + c * W, W)], sem)

        def w_wait(c, buf, sem):
            pltpu.make_async_copy(
                buf, out_hbm.at[pl.ds(base + c * W, W)], sem
            ).wait()

        g_start(0, buf0, g0)
        g_start(1, buf1, g1)

        @pl.loop(0, n_chunks - 2, step=2)
        def _(c):
            g_wait(c, buf0, g0)
            w_start(c, buf0, w0)
            g_wait(c + 1, buf1, g1)
            w_start(c + 1, buf1, w1)
            w_wait(c, buf0, w0)
            g_start(c + 2, buf0, g0)
            w_wait(c + 1, buf1, w1)
            g_start(c + 3, buf1, g1)

        cl = n_chunks - 2
        g_wait(cl, buf0, g0)
        w_start(cl, buf0, w0)
        g_wait(cl + 1, buf1, g1)
        w_start(cl + 1, buf1, w1)
        w_wait(cl, buf0, w0)
        w_wait(cl + 1, buf1, w1)

    return k(table_p, idx_p)


def kernel(idx, token_emb):
    B, T_ = idx.shape
    idx32 = idx.astype(jnp.int32)
    pad_ix = (
        jnp.arange(B, dtype=jnp.int32)[:, None] * (TP - T)
        + jnp.arange(TP - T, dtype=jnp.int32)[None, :]
    ) % V
    idx_p = jnp.concatenate([idx32, pad_ix], axis=1).reshape(-1)
    table_p = jnp.pad(token_emb, ((0, 0), (0, DP - D)))
    y = _gather_main(table_p, idx_p, B).reshape(B, TP, DP)
    return y[:, :T, :D]
